# baseline (speedup 1.0000x reference)
import functools

import jax
import jax.numpy as jnp
from jax import lax
from jax.experimental import pallas as pl
from jax.experimental.pallas import tpu as pltpu
from jax.experimental.pallas import tpu_sc as plsc

N_TRACKS_C = 1000000
D_MODEL_C = 64
BATCH_C = 16384

NUM_CORES = 2
NUM_SUBCORES = 16
LANES = 16
NUM_WORKERS = NUM_CORES * NUM_SUBCORES
B_PER_W = BATCH_C // NUM_WORKERS
CHUNK = 256
CGROUPS = CHUNK // LANES


def _body(users_hbm, tracks_hbm, first_hbm, ub0_hbm, ub1_hbm, table_hbm,
          out_hbm,
          uidx_v, tidx_v, fidx_v, thalf_v, fhalf_v, t_rows, f_rows,
          ub0_v, ub1_v, out_v, sem_t, sem_f, sem_u0, sem_u1):
    wid = lax.axis_index("s") * NUM_CORES + lax.axis_index("c")
    base = wid * B_PER_W

    pltpu.sync_copy(tracks_hbm.at[pl.ds(base, B_PER_W)], tidx_v)
    pltpu.sync_copy(first_hbm.at[pl.ds(base, B_PER_W)], fidx_v)
    pltpu.sync_copy(users_hbm.at[pl.ds(base, B_PER_W)], uidx_v)

    c0 = pltpu.async_copy(ub0_hbm.at[uidx_v], ub0_v, sem_u0)
    c1 = pltpu.async_copy(ub1_hbm.at[uidx_v], ub1_v, sem_u1)

    iota = lax.iota(jnp.int32, LANES)

    def h_body(g, carry):
        sl = pl.ds(g * LANES, LANES)
        thalf_v[sl] = tidx_v[sl] >> 1
        fhalf_v[sl] = fidx_v[sl] >> 1
        return carry

    lax.fori_loop(0, B_PER_W // LANES, h_body, 0)

    c0.wait()
    c1.wait()

    for c in range(B_PER_W // CHUNK):
        ct = pltpu.async_copy(
            table_hbm.at[thalf_v.at[pl.ds(c * CHUNK, CHUNK)]], t_rows, sem_t)
        cf = pltpu.async_copy(
            table_hbm.at[fhalf_v.at[pl.ds(c * CHUNK, CHUNK)]], f_rows, sem_f)
        ct.wait()
        cf.wait()

        def group_body(g, carry):
            lrows = g * LANES + iota
            gsl = pl.ds(c * CHUNK + g * LANES, LANES)
            tcol = (tidx_v[gsl] & 1) * D_MODEL_C
            fcol = (fidx_v[gsl] & 1) * D_MODEL_C

            def d_body(d, acc):
                fv = plsc.load_gather(f_rows, [lrows, fcol + d])
                tv = plsc.load_gather(t_rows, [lrows, tcol + d])
                return acc + fv * tv

            acc = lax.fori_loop(0, D_MODEL_C, d_body,
                                jnp.zeros((LANES,), jnp.float32))
            b0 = ub0_v[gsl]
            b1 = ub1_v[gsl]
            x = (acc - b0) * b1
            y = 1.0 / (1.0 + jnp.exp(-x))
            out_v[gsl] = y
            return carry

        lax.fori_loop(0, CGROUPS, group_body, 0)

    pltpu.sync_copy(out_v, out_hbm.at[pl.ds(base, B_PER_W)])


@jax.jit
def _run(users, tracks, first_tracks, ub0, ub1, table2):
    mesh = plsc.VectorSubcoreMesh(core_axis_name="c", subcore_axis_name="s")
    f = functools.partial(
        pl.kernel,
        out_type=jax.ShapeDtypeStruct((BATCH_C,), jnp.float32),
        mesh=mesh,
        compiler_params=pltpu.CompilerParams(
            needs_layout_passes=False, use_tc_tiling_on_sc=False),
        scratch_types=[
            pltpu.VMEM((B_PER_W,), jnp.int32),
            pltpu.VMEM((B_PER_W,), jnp.int32),
            pltpu.VMEM((B_PER_W,), jnp.int32),
            pltpu.VMEM((B_PER_W,), jnp.int32),
            pltpu.VMEM((B_PER_W,), jnp.int32),
            pltpu.VMEM((CHUNK, 2 * D_MODEL_C), jnp.float32),
            pltpu.VMEM((CHUNK, 2 * D_MODEL_C), jnp.float32),
            pltpu.VMEM((B_PER_W,), jnp.float32),
            pltpu.VMEM((B_PER_W,), jnp.float32),
            pltpu.VMEM((B_PER_W,), jnp.float32),
            pltpu.SemaphoreType.DMA,
            pltpu.SemaphoreType.DMA,
            pltpu.SemaphoreType.DMA,
            pltpu.SemaphoreType.DMA,
        ],
    )(_body)
    return f(users, tracks, first_tracks, ub0, ub1, table2)


def kernel(users, tracks, first_tracks, user_bias, tracks_table):
    users = users.astype(jnp.int32)
    tracks = tracks.astype(jnp.int32)
    first_tracks = first_tracks.astype(jnp.int32)
    ub0 = user_bias[0]
    ub1 = user_bias[1]
    table2 = tracks_table.reshape(N_TRACKS_C // 2, 2 * D_MODEL_C)
    return _run(users, tracks, first_tracks, ub0, ub1, table2)


# speedup vs baseline: 1.4308x; 1.4308x over previous
import functools

import jax
import jax.numpy as jnp
from jax import lax
from jax.experimental import pallas as pl
from jax.experimental.pallas import tpu as pltpu
from jax.experimental.pallas import tpu_sc as plsc

N_TRACKS_C = 1000000
D_MODEL_C = 64
BATCH_C = 16384

NUM_CORES = 2
NUM_SUBCORES = 16
LANES = 16
NUM_WORKERS = NUM_CORES * NUM_SUBCORES
B_PER_W = BATCH_C // NUM_WORKERS
GROUPS = B_PER_W // LANES


def _body(users_hbm, tracks_hbm, first_hbm, ub0_hbm, ub1_hbm, table_hbm,
          out_hbm,
          uidx_v, tidx_v, fidx_v, t_blk, f_blk, ub0_v, ub1_v, out_v,
          sem_t, sem_f, sem_u0, sem_u1):
    wid = lax.axis_index("s") * NUM_CORES + lax.axis_index("c")
    base = wid * B_PER_W

    pltpu.sync_copy(tracks_hbm.at[pl.ds(base, B_PER_W)], tidx_v)
    pltpu.sync_copy(first_hbm.at[pl.ds(base, B_PER_W)], fidx_v)
    pltpu.sync_copy(users_hbm.at[pl.ds(base, B_PER_W)], uidx_v)

    c0 = pltpu.async_copy(ub0_hbm.at[uidx_v], ub0_v, sem_u0)
    c1 = pltpu.async_copy(ub1_hbm.at[uidx_v], ub1_v, sem_u1)

    iota = lax.iota(jnp.int32, LANES)
    c0.wait()
    c1.wait()

    def group_body(g, carry):
        gsl = pl.ds(g * LANES, LANES)
        tv = tidx_v[gsl]
        fv = fidx_v[gsl]
        copies = []
        for j in range(LANES):
            trow = pl.multiple_of((tv[j] >> 3) * 8, 8)
            frow = pl.multiple_of((fv[j] >> 3) * 8, 8)
            copies.append(pltpu.async_copy(
                table_hbm.at[pl.ds(trow, 8), :],
                t_blk.at[pl.ds(j * 8, 8), :], sem_t))
            copies.append(pltpu.async_copy(
                table_hbm.at[pl.ds(frow, 8), :],
                f_blk.at[pl.ds(j * 8, 8), :], sem_f))
        for c in copies:
            c.wait()

        tsub = (tv & 7) + iota * 8
        fsub = (fv & 7) + iota * 8

        def d_body(d, acc):
            dv = jnp.full((LANES,), d, jnp.int32)
            fvv = plsc.load_gather(f_blk, [fsub, dv])
            tvv = plsc.load_gather(t_blk, [tsub, dv])
            return acc + fvv * tvv

        acc = lax.fori_loop(0, D_MODEL_C, d_body,
                            jnp.zeros((LANES,), jnp.float32))
        b0 = ub0_v[gsl]
        b1 = ub1_v[gsl]
        x = (acc - b0) * b1
        y = 1.0 / (1.0 + jnp.exp(-x))
        out_v[gsl] = y
        return carry

    lax.fori_loop(0, GROUPS, group_body, 0)
    pltpu.sync_copy(out_v, out_hbm.at[pl.ds(base, B_PER_W)])


@jax.jit
def _run(users, tracks, first_tracks, ub0, ub1, tracks_table):
    mesh = plsc.VectorSubcoreMesh(core_axis_name="c", subcore_axis_name="s")
    f = functools.partial(
        pl.kernel,
        out_type=jax.ShapeDtypeStruct((BATCH_C,), jnp.float32),
        mesh=mesh,
        compiler_params=pltpu.CompilerParams(
            needs_layout_passes=False, use_tc_tiling_on_sc=True),
        scratch_types=[
            pltpu.VMEM((B_PER_W,), jnp.int32),
            pltpu.VMEM((B_PER_W,), jnp.int32),
            pltpu.VMEM((B_PER_W,), jnp.int32),
            pltpu.VMEM((LANES * 8, D_MODEL_C), jnp.float32),
            pltpu.VMEM((LANES * 8, D_MODEL_C), jnp.float32),
            pltpu.VMEM((B_PER_W,), jnp.float32),
            pltpu.VMEM((B_PER_W,), jnp.float32),
            pltpu.VMEM((B_PER_W,), jnp.float32),
            pltpu.SemaphoreType.DMA,
            pltpu.SemaphoreType.DMA,
            pltpu.SemaphoreType.DMA,
            pltpu.SemaphoreType.DMA,
        ],
    )(_body)
    return f(users, tracks, first_tracks, ub0, ub1, tracks_table)


def kernel(users, tracks, first_tracks, user_bias, tracks_table):
    users = users.astype(jnp.int32)
    tracks = tracks.astype(jnp.int32)
    first_tracks = first_tracks.astype(jnp.int32)
    ub0 = user_bias[0]
    ub1 = user_bias[1]
    return _run(users, tracks, first_tracks, ub0, ub1, tracks_table)


# speedup vs baseline: 1.5637x; 1.0929x over previous
import functools

import jax
import jax.numpy as jnp
from jax import lax
from jax.experimental import pallas as pl
from jax.experimental.pallas import tpu as pltpu
from jax.experimental.pallas import tpu_sc as plsc

N_TRACKS_C = 1000000
D_MODEL_C = 64
BATCH_C = 16384

NUM_CORES = 2
NUM_SUBCORES = 16
LANES = 16
NUM_WORKERS = NUM_CORES * NUM_SUBCORES
B_PER_W = BATCH_C // NUM_WORKERS
GROUPS = B_PER_W // LANES


def _body(users_hbm, tracks_hbm, first_hbm, ub0_hbm, ub1_hbm, table_hbm,
          out_hbm,
          uidx_v, tidx_v, fidx_v, t_blk, f_blk, ub0_v, ub1_v, out_v,
          sem_t, sem_f, sem_t2, sem_f2, sem_u0, sem_u1):
    wid = lax.axis_index("s") * NUM_CORES + lax.axis_index("c")
    base = wid * B_PER_W

    pltpu.sync_copy(tracks_hbm.at[pl.ds(base, B_PER_W)], tidx_v)
    pltpu.sync_copy(first_hbm.at[pl.ds(base, B_PER_W)], fidx_v)
    pltpu.sync_copy(users_hbm.at[pl.ds(base, B_PER_W)], uidx_v)

    c0 = pltpu.async_copy(ub0_hbm.at[uidx_v], ub0_v, sem_u0)
    c1 = pltpu.async_copy(ub1_hbm.at[uidx_v], ub1_v, sem_u1)

    iota = lax.iota(jnp.int32, LANES)
    c0.wait()
    c1.wait()

    sem_tp = [sem_t, sem_t2]
    sem_fp = [sem_f, sem_f2]

    def issue(g, p):
        gsl = pl.ds(g * LANES, LANES)
        tv = tidx_v[gsl]
        fv = fidx_v[gsl]
        for j in range(LANES):
            trow = pl.multiple_of((tv[j] >> 3) * 8, 8)
            frow = pl.multiple_of((fv[j] >> 3) * 8, 8)
            pltpu.async_copy(
                table_hbm.at[pl.ds(trow, 8), :],
                t_blk.at[p, pl.ds(j * 8, 8), :], sem_tp[p])
            pltpu.async_copy(
                table_hbm.at[pl.ds(frow, 8), :],
                f_blk.at[p, pl.ds(j * 8, 8), :], sem_fp[p])

    def drain_compute(g, p):
        pltpu.make_async_copy(
            table_hbm.at[pl.ds(0, LANES * 8), :],
            t_blk.at[p], sem_tp[p]).wait()
        pltpu.make_async_copy(
            table_hbm.at[pl.ds(0, LANES * 8), :],
            f_blk.at[p], sem_fp[p]).wait()
        gsl = pl.ds(g * LANES, LANES)
        tsub = (tidx_v[gsl] & 7) + iota * 8
        fsub = (fidx_v[gsl] & 7) + iota * 8
        pv = jnp.full((LANES,), p, jnp.int32)

        def d_body(dd, acc):
            for k in range(4):
                dv = jnp.full((LANES,), dd * 4 + k, jnp.int32)
                fvv = plsc.load_gather(f_blk, [pv, fsub, dv])
                tvv = plsc.load_gather(t_blk, [pv, tsub, dv])
                acc = acc + fvv * tvv
            return acc

        acc = lax.fori_loop(0, D_MODEL_C // 4, d_body,
                            jnp.zeros((LANES,), jnp.float32))
        b0 = ub0_v[gsl]
        b1 = ub1_v[gsl]
        x = (acc - b0) * b1
        y = 1.0 / (1.0 + jnp.exp(-x))
        out_v[gsl] = y

    issue(0, 0)
    issue(1, 1)

    def pipe_body(h, carry):
        g = h * 2
        drain_compute(g, 0)

        @pl.when(g + 2 < GROUPS)
        def _():
            issue(g + 2, 0)

        drain_compute(g + 1, 1)

        @pl.when(g + 3 < GROUPS)
        def _():
            issue(g + 3, 1)

        return carry

    lax.fori_loop(0, GROUPS // 2, pipe_body, 0)
    pltpu.sync_copy(out_v, out_hbm.at[pl.ds(base, B_PER_W)])


@jax.jit
def _run(users, tracks, first_tracks, ub0, ub1, tracks_table):
    mesh = plsc.VectorSubcoreMesh(core_axis_name="c", subcore_axis_name="s")
    f = functools.partial(
        pl.kernel,
        out_type=jax.ShapeDtypeStruct((BATCH_C,), jnp.float32),
        mesh=mesh,
        compiler_params=pltpu.CompilerParams(
            needs_layout_passes=False, use_tc_tiling_on_sc=True),
        scratch_types=[
            pltpu.VMEM((B_PER_W,), jnp.int32),
            pltpu.VMEM((B_PER_W,), jnp.int32),
            pltpu.VMEM((B_PER_W,), jnp.int32),
            pltpu.VMEM((2, LANES * 8, D_MODEL_C), jnp.float32),
            pltpu.VMEM((2, LANES * 8, D_MODEL_C), jnp.float32),
            pltpu.VMEM((B_PER_W,), jnp.float32),
            pltpu.VMEM((B_PER_W,), jnp.float32),
            pltpu.VMEM((B_PER_W,), jnp.float32),
            pltpu.SemaphoreType.DMA,
            pltpu.SemaphoreType.DMA,
            pltpu.SemaphoreType.DMA,
            pltpu.SemaphoreType.DMA,
            pltpu.SemaphoreType.DMA,
            pltpu.SemaphoreType.DMA,
        ],
    )(_body)
    return f(users, tracks, first_tracks, ub0, ub1, tracks_table)


def kernel(users, tracks, first_tracks, user_bias, tracks_table):
    users = users.astype(jnp.int32)
    tracks = tracks.astype(jnp.int32)
    first_tracks = first_tracks.astype(jnp.int32)
    ub0 = user_bias[0]
    ub1 = user_bias[1]
    return _run(users, tracks, first_tracks, ub0, ub1, tracks_table)


# speedup vs baseline: 1.5702x; 1.0042x over previous
import functools

import jax
import jax.numpy as jnp
from jax import lax
from jax.experimental import pallas as pl
from jax.experimental.pallas import tpu as pltpu
from jax.experimental.pallas import tpu_sc as plsc

N_TRACKS_C = 1000000
D_MODEL_C = 64
BATCH_C = 16384

NUM_CORES = 2
NUM_SUBCORES = 16
LANES = 16
NUM_WORKERS = NUM_CORES * NUM_SUBCORES
B_PER_W = BATCH_C // NUM_WORKERS
GROUPS = B_PER_W // LANES


def _body(users_hbm, tracks_hbm, first_hbm, ub0_hbm, ub1_hbm, table_hbm,
          out_hbm,
          uidx_v, tidx_v, fidx_v, t_blk, f_blk, ub0_v, ub1_v, out_v,
          sem_t, sem_f, sem_t2, sem_f2, sem_u0, sem_u1):
    wid = lax.axis_index("s") * NUM_CORES + lax.axis_index("c")
    base = wid * B_PER_W

    pltpu.sync_copy(tracks_hbm.at[pl.ds(base, B_PER_W)], tidx_v)
    pltpu.sync_copy(first_hbm.at[pl.ds(base, B_PER_W)], fidx_v)
    pltpu.sync_copy(users_hbm.at[pl.ds(base, B_PER_W)], uidx_v)

    c0 = pltpu.async_copy(ub0_hbm.at[uidx_v], ub0_v, sem_u0)
    c1 = pltpu.async_copy(ub1_hbm.at[uidx_v], ub1_v, sem_u1)

    iota = lax.iota(jnp.int32, LANES)
    c0.wait()
    c1.wait()

    sem_tp = [sem_t, sem_t2]
    sem_fp = [sem_f, sem_f2]

    def issue(g, p):
        gsl = pl.ds(g * LANES, LANES)
        tv = tidx_v[gsl]
        fv = fidx_v[gsl]
        for j in range(LANES):
            trow = pl.multiple_of((tv[j] >> 3) * 8, 8)
            frow = pl.multiple_of((fv[j] >> 3) * 8, 8)
            pltpu.async_copy(
                table_hbm.at[pl.ds(trow, 8), :],
                t_blk.at[p, pl.ds(j * 8, 8), :], sem_tp[p])
            pltpu.async_copy(
                table_hbm.at[pl.ds(frow, 8), :],
                f_blk.at[p, pl.ds(j * 8, 8), :], sem_fp[p])

    def drain_compute(g, p):
        pltpu.make_async_copy(
            table_hbm.at[pl.ds(0, LANES * 8), :],
            t_blk.at[p], sem_tp[p]).wait()
        pltpu.make_async_copy(
            table_hbm.at[pl.ds(0, LANES * 8), :],
            f_blk.at[p], sem_fp[p]).wait()
        gsl = pl.ds(g * LANES, LANES)
        tsub = (tidx_v[gsl] & 7) + iota * 8
        fsub = (fidx_v[gsl] & 7) + iota * 8
        t_b = t_blk.at[p]
        f_b = f_blk.at[p]

        def d_body(dd, acc):
            for k in range(4):
                dv = jnp.full((LANES,), dd * 4 + k, jnp.int32)
                fvv = plsc.load_gather(f_b, [fsub, dv])
                tvv = plsc.load_gather(t_b, [tsub, dv])
                acc = acc + fvv * tvv
            return acc

        acc = lax.fori_loop(0, D_MODEL_C // 4, d_body,
                            jnp.zeros((LANES,), jnp.float32))
        b0 = ub0_v[gsl]
        b1 = ub1_v[gsl]
        x = (acc - b0) * b1
        y = 1.0 / (1.0 + jnp.exp(-x))
        out_v[gsl] = y

    issue(0, 0)
    issue(1, 1)

    def pipe_body(h, carry):
        g = h * 2
        drain_compute(g, 0)

        @pl.when(g + 2 < GROUPS)
        def _():
            issue(g + 2, 0)

        drain_compute(g + 1, 1)

        @pl.when(g + 3 < GROUPS)
        def _():
            issue(g + 3, 1)

        return carry

    lax.fori_loop(0, GROUPS // 2, pipe_body, 0)
    pltpu.sync_copy(out_v, out_hbm.at[pl.ds(base, B_PER_W)])


@jax.jit
def _run(users, tracks, first_tracks, ub0, ub1, tracks_table):
    mesh = plsc.VectorSubcoreMesh(core_axis_name="c", subcore_axis_name="s")
    f = functools.partial(
        pl.kernel,
        out_type=jax.ShapeDtypeStruct((BATCH_C,), jnp.float32),
        mesh=mesh,
        compiler_params=pltpu.CompilerParams(
            needs_layout_passes=False, use_tc_tiling_on_sc=True),
        scratch_types=[
            pltpu.VMEM((B_PER_W,), jnp.int32),
            pltpu.VMEM((B_PER_W,), jnp.int32),
            pltpu.VMEM((B_PER_W,), jnp.int32),
            pltpu.VMEM((2, LANES * 8, D_MODEL_C), jnp.float32),
            pltpu.VMEM((2, LANES * 8, D_MODEL_C), jnp.float32),
            pltpu.VMEM((B_PER_W,), jnp.float32),
            pltpu.VMEM((B_PER_W,), jnp.float32),
            pltpu.VMEM((B_PER_W,), jnp.float32),
            pltpu.SemaphoreType.DMA,
            pltpu.SemaphoreType.DMA,
            pltpu.SemaphoreType.DMA,
            pltpu.SemaphoreType.DMA,
            pltpu.SemaphoreType.DMA,
            pltpu.SemaphoreType.DMA,
        ],
    )(_body)
    return f(users, tracks, first_tracks, ub0, ub1, tracks_table)


def kernel(users, tracks, first_tracks, user_bias, tracks_table):
    users = users.astype(jnp.int32)
    tracks = tracks.astype(jnp.int32)
    first_tracks = first_tracks.astype(jnp.int32)
    ub0 = user_bias[0]
    ub1 = user_bias[1]
    return _run(users, tracks, first_tracks, ub0, ub1, tracks_table)
